# baseline (speedup 1.0000x reference)
import functools

import jax
import jax.numpy as jnp
from jax import lax
from jax.experimental import pallas as pl
from jax.experimental.pallas import tpu as pltpu
from jax.experimental.pallas import tpu_sc as plsc

N = 10000
E = 320000
D = 128
K = 4

NUM_CORES = 2
NUM_TILES = 16
EB = 80
ROWS_PER_TILE = N // NUM_TILES
EDGES_PER_TILE_A = E // (NUM_CORES * NUM_TILES)
NB_A = EDGES_PER_TILE_A // EB
EDGES_PER_TILE_C = E // NUM_TILES
NB_C = EDGES_PER_TILE_C // EB

NBLK = 400
NGRID = N // NBLK


def _mesh():
    return plsc.VectorSubcoreMesh(core_axis_name="c", subcore_axis_name="s")


def _sc_params():
    return pltpu.CompilerParams(use_tc_tiling_on_sc=False)


def _deg_body(col_hbm, deg_hbm, acc, idx_v, ones_v, zrow_v):
    c = lax.axis_index("c")
    s = lax.axis_index("s")
    one16 = jnp.full((16,), 1.0, jnp.float32)
    zero16 = jnp.zeros((16,), jnp.float32)
    for r in range(EB):
        ones_v[r, :] = one16
    for r in range(NB_A):
        zrow_v[r, :] = zero16
    for j in range(ROWS_PER_TILE // NB_A):
        pltpu.sync_copy(zrow_v, acc.at[pl.ds(s * ROWS_PER_TILE + j * NB_A, NB_A)])
    plsc.subcore_barrier()
    base_row = c * (NUM_TILES * NB_A) + s * NB_A
    pltpu.sync_copy(col_hbm.at[pl.ds(base_row, NB_A)], idx_v)

    def body(i, carry):
        pltpu.sync_copy(ones_v, acc.at[idx_v.at[i]], add=True)
        return carry

    lax.fori_loop(0, NB_A, body, 0)
    plsc.subcore_barrier()
    pltpu.sync_copy(acc.at[pl.ds(s * ROWS_PER_TILE, ROWS_PER_TILE)],
                    deg_hbm.at[c, pl.ds(s * ROWS_PER_TILE, ROWS_PER_TILE)])


def _deg_partials(col2d):
    fn = pl.kernel(
        _deg_body,
        out_type=jax.ShapeDtypeStruct((NUM_CORES, N, 16), jnp.float32),
        mesh=_mesh(),
        compiler_params=_sc_params(),
        scratch_types=[
            pltpu.VMEM_SHARED((N, 16), jnp.float32),
            pltpu.VMEM((NB_A, EB), jnp.int32),
            pltpu.VMEM((EB, 16), jnp.float32),
            pltpu.VMEM((NB_A, 16), jnp.float32),
        ],
    )
    return fn(col2d)


def _zdis_body(inp_ref, w_ref, degp_ref, z_ref, dis_ref):
    d = degp_ref[...]
    deg = jnp.sum(d[0] + d[1], axis=1, keepdims=True) * (1.0 / 16.0) + 1.0
    dis = lax.rsqrt(deg)
    w = w_ref[...]
    for k in range(K):
        x = inp_ref[:, k * D:(k + 1) * D]
        y = jnp.dot(x, w, preferred_element_type=jnp.float32)
        z_ref[k, :, :] = y * dis
    dis_ref[...] = dis


def _z_and_dis(inputs2d, W, deg_part):
    return pl.pallas_call(
        _zdis_body,
        grid=(NGRID,),
        in_specs=[
            pl.BlockSpec((NBLK, K * D), lambda i: (i, 0)),
            pl.BlockSpec((D, D), lambda i: (0, 0)),
            pl.BlockSpec((NUM_CORES, NBLK, 16), lambda i: (0, i, 0)),
        ],
        out_specs=[
            pl.BlockSpec((K, NBLK, D), lambda i: (0, i, 0)),
            pl.BlockSpec((NBLK, 1), lambda i: (i, 0)),
        ],
        out_shape=[
            jax.ShapeDtypeStruct((K, N, D), jnp.float32),
            jax.ShapeDtypeStruct((N, 1), jnp.float32),
        ],
    )(inputs2d, W, deg_part)


CH = 50
NBUF = 4


def _agg_body(z_hbm, row_hbm, col_hbm, agg_hbm, acc, rowi_v, coli_v,
              r0, r1, r2, r3, sg0, sg1, sg2, sg3, ss0, ss1, ss2, ss3):
    c = lax.axis_index("c")
    s = lax.axis_index("s")
    bufs = (r0, r1, r2, r3)
    gsem = (sg0, sg1, sg2, sg3)
    ssem = (ss0, ss1, ss2, ss3)

    def run_chunk(kk, batch0, nb):
        pltpu.sync_copy(row_hbm.at[pl.ds(batch0, nb)], rowi_v.at[pl.ds(0, nb)])
        pltpu.sync_copy(col_hbm.at[pl.ds(batch0, nb)], coli_v.at[pl.ds(0, nb)])
        zk = z_hbm.at[kk]

        def g_start(g, b):
            pltpu.async_copy(zk.at[rowi_v.at[g]], bufs[b], gsem[b])

        def g_wait(g, b):
            pltpu.make_async_copy(zk.at[rowi_v.at[g]], bufs[b], gsem[b]).wait()

        def s_start(g, b):
            pltpu.async_copy(bufs[b], acc.at[coli_v.at[g]], ssem[b], add=True)

        def s_wait(g, b):
            pltpu.make_async_copy(bufs[b], acc.at[coli_v.at[g]], ssem[b]).wait()

        def emit(g, b, wait_prev, start_next):
            g_wait(g, b)
            s_start(g, b)
            if wait_prev:
                s_wait(g - 2, (b + 2) % NBUF)
            if start_next:
                g_start(g + 2, (b + 2) % NBUF)

        g_start(0, 0)
        g_start(1, 1)
        emit(0, 0, False, nb > 2)
        emit(1, 1, False, nb > 3)
        m4 = ((nb - 4) // NBUF) * NBUF

        def pipe(p, carry):
            g = 2 + NBUF * p
            for j in range(NBUF):
                emit(g + j, (2 + j) % NBUF, True, True)
            return carry

        lax.fori_loop(0, m4 // NBUF, pipe, 0)
        for g in range(2 + m4, nb):
            emit(g, g % NBUF, True, g + 2 < nb)
        s_wait(nb - 2, (nb - 2) % NBUF)
        s_wait(nb - 1, (nb - 1) % NBUF)

    for sl in range(2):
        kk = 2 * c + sl
        pltpu.sync_copy(z_hbm.at[kk, pl.ds(s * ROWS_PER_TILE, ROWS_PER_TILE)],
                        acc.at[pl.ds(s * ROWS_PER_TILE, ROWS_PER_TILE)])
        plsc.subcore_barrier()
        for ci in range(NB_C // CH):
            run_chunk(kk, s * NB_C + ci * CH, CH)
        plsc.subcore_barrier()
        pltpu.sync_copy(acc.at[pl.ds(s * ROWS_PER_TILE, ROWS_PER_TILE)],
                        agg_hbm.at[kk, pl.ds(s * ROWS_PER_TILE, ROWS_PER_TILE)])


def _aggregate(z, row2d, col2d):
    fn = pl.kernel(
        _agg_body,
        out_type=jax.ShapeDtypeStruct((K, N, D), jnp.float32),
        mesh=_mesh(),
        compiler_params=_sc_params(),
        scratch_types=[
            pltpu.VMEM_SHARED((N, D), jnp.float32),
            pltpu.VMEM((CH, EB), jnp.int32),
            pltpu.VMEM((CH, EB), jnp.int32),
            pltpu.VMEM((EB, D), jnp.float32),
            pltpu.VMEM((EB, D), jnp.float32),
            pltpu.VMEM((EB, D), jnp.float32),
            pltpu.VMEM((EB, D), jnp.float32),
            pltpu.SemaphoreType.DMA,
            pltpu.SemaphoreType.DMA,
            pltpu.SemaphoreType.DMA,
            pltpu.SemaphoreType.DMA,
            pltpu.SemaphoreType.DMA,
            pltpu.SemaphoreType.DMA,
            pltpu.SemaphoreType.DMA,
            pltpu.SemaphoreType.DMA,
        ],
    )
    return fn(z, row2d, col2d)


def _out_body(agg_ref, dis_ref, b_ref, o_ref):
    dis = dis_ref[...]
    bb = b_ref[...]
    for k in range(K):
        o_ref[:, k, :] = jnp.maximum(agg_ref[k, :, :] * dis + bb, 0.0)


def _finalize(agg3, dis, b2d):
    return pl.pallas_call(
        _out_body,
        grid=(NGRID,),
        in_specs=[
            pl.BlockSpec((K, NBLK, D), lambda i: (0, i, 0)),
            pl.BlockSpec((NBLK, 1), lambda i: (i, 0)),
            pl.BlockSpec((1, D), lambda i: (0, 0)),
        ],
        out_specs=pl.BlockSpec((NBLK, K, D), lambda i: (i, 0, 0)),
        out_shape=jax.ShapeDtypeStruct((N, K, D), jnp.float32),
    )(agg3, dis, b2d)


def kernel(inputs, edge_index, W, b):
    row2d = edge_index[0].reshape(E // EB, EB)
    col2d = edge_index[1].reshape(E // EB, EB)
    deg_part = _deg_partials(col2d)
    z3, dis = _z_and_dis(inputs.reshape(N, K * D), W, deg_part)
    agg = _aggregate(z3, row2d, col2d)
    return _finalize(agg, dis, b.reshape(1, D))


# speedup vs baseline: 1.1495x; 1.1495x over previous
import functools

import jax
import jax.numpy as jnp
from jax import lax
from jax.experimental import pallas as pl
from jax.experimental.pallas import tpu as pltpu
from jax.experimental.pallas import tpu_sc as plsc

N = 10000
E = 320000
D = 128
K = 4

NUM_CORES = 2
NUM_TILES = 16
EB = 80
ROWS_PER_TILE = N // NUM_TILES
EDGES_PER_TILE_A = E // (NUM_CORES * NUM_TILES)
NB_A = EDGES_PER_TILE_A // EB
EDGES_PER_TILE_C = E // NUM_TILES
NB_C = EDGES_PER_TILE_C // EB

NBLK = 400
NGRID = N // NBLK


def _mesh():
    return plsc.VectorSubcoreMesh(core_axis_name="c", subcore_axis_name="s")


def _sc_params():
    return pltpu.CompilerParams(use_tc_tiling_on_sc=False)


def _deg_body(col_hbm, deg_hbm, acc, idx_v, ones_v, zrow_v):
    c = lax.axis_index("c")
    s = lax.axis_index("s")
    one16 = jnp.full((16,), 1.0, jnp.float32)
    zero16 = jnp.zeros((16,), jnp.float32)
    for r in range(EB):
        ones_v[r, :] = one16
    for r in range(NB_A):
        zrow_v[r, :] = zero16
    for j in range(ROWS_PER_TILE // NB_A):
        pltpu.sync_copy(zrow_v, acc.at[pl.ds(s * ROWS_PER_TILE + j * NB_A, NB_A)])
    plsc.subcore_barrier()
    base_row = c * (NUM_TILES * NB_A) + s * NB_A
    pltpu.sync_copy(col_hbm.at[pl.ds(base_row, NB_A)], idx_v)

    def body(i, carry):
        pltpu.sync_copy(ones_v, acc.at[idx_v.at[i]], add=True)
        return carry

    lax.fori_loop(0, NB_A, body, 0)
    plsc.subcore_barrier()
    pltpu.sync_copy(acc.at[pl.ds(s * ROWS_PER_TILE, ROWS_PER_TILE)],
                    deg_hbm.at[c, pl.ds(s * ROWS_PER_TILE, ROWS_PER_TILE)])


def _deg_partials(col2d):
    fn = pl.kernel(
        _deg_body,
        out_type=jax.ShapeDtypeStruct((NUM_CORES, N, 16), jnp.float32),
        mesh=_mesh(),
        compiler_params=_sc_params(),
        scratch_types=[
            pltpu.VMEM_SHARED((N, 16), jnp.float32),
            pltpu.VMEM((NB_A, EB), jnp.int32),
            pltpu.VMEM((EB, 16), jnp.float32),
            pltpu.VMEM((NB_A, 16), jnp.float32),
        ],
    )
    return fn(col2d)


def _zdis_body(inp_ref, w_ref, degp_ref, z_ref, dis_ref):
    d = degp_ref[...]
    deg = jnp.sum(d[0] + d[1], axis=1, keepdims=True) * (1.0 / 16.0) + 1.0
    dis = lax.rsqrt(deg)
    w = w_ref[...]
    for k in range(K):
        x = inp_ref[:, k * D:(k + 1) * D]
        y = jnp.dot(x, w, preferred_element_type=jnp.float32)
        z_ref[k, :, :] = y * dis
    dis_ref[...] = dis


def _z_and_dis(inputs2d, W, deg_part):
    return pl.pallas_call(
        _zdis_body,
        grid=(NGRID,),
        in_specs=[
            pl.BlockSpec((NBLK, K * D), lambda i: (i, 0)),
            pl.BlockSpec((D, D), lambda i: (0, 0)),
            pl.BlockSpec((NUM_CORES, NBLK, 16), lambda i: (0, i, 0)),
        ],
        out_specs=[
            pl.BlockSpec((K, NBLK, D), lambda i: (0, i, 0)),
            pl.BlockSpec((NBLK, 1), lambda i: (i, 0)),
        ],
        out_shape=[
            jax.ShapeDtypeStruct((K, N, D), jnp.float32),
            jax.ShapeDtypeStruct((N, 1), jnp.float32),
        ],
    )(inputs2d, W, deg_part)


CH = 125
NBUF = 3


def _agg_body(z_hbm, row_hbm, col_hbm, agg_hbm, acc, rowi_v, coli_v,
              r0, r1, r2, sg0, sg1, sg2, ss0, ss1, ss2):
    c = lax.axis_index("c")
    s = lax.axis_index("s")
    bufs = (r0, r1, r2)
    gsem = (sg0, sg1, sg2)
    ssem = (ss0, ss1, ss2)

    def run_chunk(kk, batch0, nb):
        pltpu.sync_copy(row_hbm.at[pl.ds(batch0, nb)], rowi_v.at[pl.ds(0, nb)])
        pltpu.sync_copy(col_hbm.at[pl.ds(batch0, nb)], coli_v.at[pl.ds(0, nb)])
        zk = z_hbm.at[kk]

        def g_start(g, b):
            pltpu.async_copy(zk.at[rowi_v.at[g]], bufs[b], gsem[b])

        def g_wait(g, b):
            pltpu.make_async_copy(zk.at[rowi_v.at[g]], bufs[b], gsem[b]).wait()

        def s_start(g, b):
            pltpu.async_copy(bufs[b], acc.at[coli_v.at[g]], ssem[b], add=True)

        def s_wait(g, b):
            pltpu.make_async_copy(bufs[b], acc.at[coli_v.at[g]], ssem[b]).wait()

        def emit(g, b, wait_prev, start_next):
            g_wait(g, b)
            s_start(g, b)
            if wait_prev:
                s_wait(g - 1, (b + 2) % NBUF)
            if start_next:
                g_start(g + 2, (b + 2) % NBUF)

        g_start(0, 0)
        g_start(1, 1)
        emit(0, 0, False, nb > 2)
        m3 = ((nb - 3) // NBUF) * NBUF

        def pipe(p, carry):
            g = 1 + NBUF * p
            for j in range(NBUF):
                emit(g + j, (1 + j) % NBUF, True, True)
            return carry

        lax.fori_loop(0, m3 // NBUF, pipe, 0)
        for g in range(1 + m3, nb):
            emit(g, g % NBUF, True, g + 2 < nb)
        s_wait(nb - 1, (nb - 1) % NBUF)

    for sl in range(2):
        kk = 2 * c + sl
        pltpu.sync_copy(z_hbm.at[kk, pl.ds(s * ROWS_PER_TILE, ROWS_PER_TILE)],
                        acc.at[pl.ds(s * ROWS_PER_TILE, ROWS_PER_TILE)])
        plsc.subcore_barrier()
        run_chunk(kk, s * NB_C, CH)
        run_chunk(kk, s * NB_C + CH, NB_C - CH)
        plsc.subcore_barrier()
        pltpu.sync_copy(acc.at[pl.ds(s * ROWS_PER_TILE, ROWS_PER_TILE)],
                        agg_hbm.at[kk, pl.ds(s * ROWS_PER_TILE, ROWS_PER_TILE)])


def _aggregate(z, row2d, col2d):
    fn = pl.kernel(
        _agg_body,
        out_type=jax.ShapeDtypeStruct((K, N, D), jnp.float32),
        mesh=_mesh(),
        compiler_params=_sc_params(),
        scratch_types=[
            pltpu.VMEM_SHARED((N, D), jnp.float32),
            pltpu.VMEM((CH, EB), jnp.int32),
            pltpu.VMEM((CH, EB), jnp.int32),
            pltpu.VMEM((EB, D), jnp.float32),
            pltpu.VMEM((EB, D), jnp.float32),
            pltpu.VMEM((EB, D), jnp.float32),
            pltpu.SemaphoreType.DMA,
            pltpu.SemaphoreType.DMA,
            pltpu.SemaphoreType.DMA,
            pltpu.SemaphoreType.DMA,
            pltpu.SemaphoreType.DMA,
            pltpu.SemaphoreType.DMA,
        ],
    )
    return fn(z, row2d, col2d)


def _out_body(agg_ref, dis_ref, b_ref, o_ref):
    dis = dis_ref[...]
    bb = b_ref[...]
    for k in range(K):
        o_ref[:, k, :] = jnp.maximum(agg_ref[k, :, :] * dis + bb, 0.0)


def _finalize(agg3, dis, b2d):
    return pl.pallas_call(
        _out_body,
        grid=(NGRID,),
        in_specs=[
            pl.BlockSpec((K, NBLK, D), lambda i: (0, i, 0)),
            pl.BlockSpec((NBLK, 1), lambda i: (i, 0)),
            pl.BlockSpec((1, D), lambda i: (0, 0)),
        ],
        out_specs=pl.BlockSpec((NBLK, K, D), lambda i: (i, 0, 0)),
        out_shape=jax.ShapeDtypeStruct((N, K, D), jnp.float32),
    )(agg3, dis, b2d)


def kernel(inputs, edge_index, W, b):
    row2d = edge_index[0].reshape(E // EB, EB)
    col2d = edge_index[1].reshape(E // EB, EB)
    deg_part = _deg_partials(col2d)
    z3, dis = _z_and_dis(inputs.reshape(N, K * D), W, deg_part)
    agg = _aggregate(z3, row2d, col2d)
    return _finalize(agg, dis, b.reshape(1, D))


# speedup vs baseline: 1.1622x; 1.0110x over previous
import functools

import jax
import jax.numpy as jnp
from jax import lax
from jax.experimental import pallas as pl
from jax.experimental.pallas import tpu as pltpu
from jax.experimental.pallas import tpu_sc as plsc

N = 10000
E = 320000
D = 128
K = 4

NUM_CORES = 2
NUM_TILES = 16
EB = 80
ROWS_PER_TILE = N // NUM_TILES
EDGES_PER_TILE_A = E // (NUM_CORES * NUM_TILES)
NB_A = EDGES_PER_TILE_A // EB
EDGES_PER_TILE_C = E // NUM_TILES
NB_C = EDGES_PER_TILE_C // EB

NBLK = 400
NGRID = N // NBLK


def _mesh():
    return plsc.VectorSubcoreMesh(core_axis_name="c", subcore_axis_name="s")


def _sc_params():
    return pltpu.CompilerParams(use_tc_tiling_on_sc=False)


def _deg_body(col_hbm, deg_hbm, acc, idx_v, ones_v, zrow_v):
    c = lax.axis_index("c")
    s = lax.axis_index("s")
    one16 = jnp.full((16,), 1.0, jnp.float32)
    zero16 = jnp.zeros((16,), jnp.float32)
    for r in range(EB):
        ones_v[r, :] = one16
    for r in range(NB_A):
        zrow_v[r, :] = zero16
    for j in range(ROWS_PER_TILE // NB_A):
        pltpu.sync_copy(zrow_v, acc.at[pl.ds(s * ROWS_PER_TILE + j * NB_A, NB_A)])
    plsc.subcore_barrier()
    base_row = c * (NUM_TILES * NB_A) + s * NB_A
    pltpu.sync_copy(col_hbm.at[pl.ds(base_row, NB_A)], idx_v)

    def body(i, carry):
        pltpu.sync_copy(ones_v, acc.at[idx_v.at[i]], add=True)
        return carry

    lax.fori_loop(0, NB_A, body, 0)
    plsc.subcore_barrier()
    pltpu.sync_copy(acc.at[pl.ds(s * ROWS_PER_TILE, ROWS_PER_TILE)],
                    deg_hbm.at[c, pl.ds(s * ROWS_PER_TILE, ROWS_PER_TILE)])


def _deg_partials(col2d):
    fn = pl.kernel(
        _deg_body,
        out_type=jax.ShapeDtypeStruct((NUM_CORES, N, 16), jnp.float32),
        mesh=_mesh(),
        compiler_params=_sc_params(),
        scratch_types=[
            pltpu.VMEM_SHARED((N, 16), jnp.float32),
            pltpu.VMEM((NB_A, EB), jnp.int32),
            pltpu.VMEM((EB, 16), jnp.float32),
            pltpu.VMEM((NB_A, 16), jnp.float32),
        ],
    )
    return fn(col2d)


def _mm_body(inp_ref, w_ref, y_ref):
    w = w_ref[...]
    for k in range(K):
        x = inp_ref[:, k * D:(k + 1) * D]
        y_ref[k, :, :] = jnp.dot(x, w, preferred_element_type=jnp.float32)


def _matmul(inputs2d, W):
    return pl.pallas_call(
        _mm_body,
        grid=(NGRID,),
        in_specs=[
            pl.BlockSpec((NBLK, K * D), lambda i: (i, 0)),
            pl.BlockSpec((D, D), lambda i: (0, 0)),
        ],
        out_specs=pl.BlockSpec((K, NBLK, D), lambda i: (0, i, 0)),
        out_shape=jax.ShapeDtypeStruct((K, N, D), jnp.float32),
    )(inputs2d, W)


def _zdis_body(y_ref, degp_ref, z_ref, dis_ref):
    d = degp_ref[...]
    deg = jnp.sum(d[0] + d[1], axis=1, keepdims=True) * (1.0 / 16.0) + 1.0
    dis = lax.rsqrt(deg)
    for k in range(K):
        z_ref[k, :, :] = y_ref[k, :, :] * dis
    dis_ref[...] = dis


def _z_and_dis(y, deg_part):
    return pl.pallas_call(
        _zdis_body,
        grid=(NGRID,),
        in_specs=[
            pl.BlockSpec((K, NBLK, D), lambda i: (0, i, 0)),
            pl.BlockSpec((NUM_CORES, NBLK, 16), lambda i: (0, i, 0)),
        ],
        out_specs=[
            pl.BlockSpec((K, NBLK, D), lambda i: (0, i, 0)),
            pl.BlockSpec((NBLK, 1), lambda i: (i, 0)),
        ],
        out_shape=[
            jax.ShapeDtypeStruct((K, N, D), jnp.float32),
            jax.ShapeDtypeStruct((N, 1), jnp.float32),
        ],
    )(y, deg_part)


CH = 125
NBUF = 3


def _agg_body(z_hbm, row_hbm, col_hbm, agg_hbm, acc, rowi_v, coli_v,
              r0, r1, r2, sg0, sg1, sg2, ss0, ss1, ss2):
    c = lax.axis_index("c")
    s = lax.axis_index("s")
    bufs = (r0, r1, r2)
    gsem = (sg0, sg1, sg2)
    ssem = (ss0, ss1, ss2)

    def run_chunk(kk, batch0, nb):
        pltpu.sync_copy(row_hbm.at[pl.ds(batch0, nb)], rowi_v.at[pl.ds(0, nb)])
        pltpu.sync_copy(col_hbm.at[pl.ds(batch0, nb)], coli_v.at[pl.ds(0, nb)])
        zk = z_hbm.at[kk]

        def g_start(g, b):
            pltpu.async_copy(zk.at[rowi_v.at[g]], bufs[b], gsem[b])

        def g_wait(g, b):
            pltpu.make_async_copy(zk.at[rowi_v.at[g]], bufs[b], gsem[b]).wait()

        def s_start(g, b):
            pltpu.async_copy(bufs[b], acc.at[coli_v.at[g]], ssem[b], add=True)

        def s_wait(g, b):
            pltpu.make_async_copy(bufs[b], acc.at[coli_v.at[g]], ssem[b]).wait()

        def emit(g, b, wait_prev, start_next):
            g_wait(g, b)
            s_start(g, b)
            if wait_prev:
                s_wait(g - 1, (b + 2) % NBUF)
            if start_next:
                g_start(g + 2, (b + 2) % NBUF)

        g_start(0, 0)
        g_start(1, 1)
        emit(0, 0, False, nb > 2)
        m3 = ((nb - 3) // NBUF) * NBUF

        def pipe(p, carry):
            g = 1 + NBUF * p
            for j in range(NBUF):
                emit(g + j, (1 + j) % NBUF, True, True)
            return carry

        lax.fori_loop(0, m3 // NBUF, pipe, 0)
        for g in range(1 + m3, nb):
            emit(g, g % NBUF, True, g + 2 < nb)
        s_wait(nb - 1, (nb - 1) % NBUF)

    for sl in range(2):
        kk = 2 * c + sl
        pltpu.sync_copy(z_hbm.at[kk, pl.ds(s * ROWS_PER_TILE, ROWS_PER_TILE)],
                        acc.at[pl.ds(s * ROWS_PER_TILE, ROWS_PER_TILE)])
        plsc.subcore_barrier()
        run_chunk(kk, s * NB_C, CH)
        run_chunk(kk, s * NB_C + CH, NB_C - CH)
        plsc.subcore_barrier()
        pltpu.sync_copy(acc.at[pl.ds(s * ROWS_PER_TILE, ROWS_PER_TILE)],
                        agg_hbm.at[kk, pl.ds(s * ROWS_PER_TILE, ROWS_PER_TILE)])


def _aggregate(z, row2d, col2d):
    fn = pl.kernel(
        _agg_body,
        out_type=jax.ShapeDtypeStruct((K, N, D), jnp.float32),
        mesh=_mesh(),
        compiler_params=_sc_params(),
        scratch_types=[
            pltpu.VMEM_SHARED((N, D), jnp.float32),
            pltpu.VMEM((CH, EB), jnp.int32),
            pltpu.VMEM((CH, EB), jnp.int32),
            pltpu.VMEM((EB, D), jnp.float32),
            pltpu.VMEM((EB, D), jnp.float32),
            pltpu.VMEM((EB, D), jnp.float32),
            pltpu.SemaphoreType.DMA,
            pltpu.SemaphoreType.DMA,
            pltpu.SemaphoreType.DMA,
            pltpu.SemaphoreType.DMA,
            pltpu.SemaphoreType.DMA,
            pltpu.SemaphoreType.DMA,
        ],
    )
    return fn(z, row2d, col2d)


def _out_body(agg_ref, dis_ref, b_ref, o_ref):
    dis = dis_ref[...]
    bb = b_ref[...]
    for k in range(K):
        o_ref[:, k, :] = jnp.maximum(agg_ref[k, :, :] * dis + bb, 0.0)


def _finalize(agg3, dis, b2d):
    return pl.pallas_call(
        _out_body,
        grid=(NGRID,),
        in_specs=[
            pl.BlockSpec((K, NBLK, D), lambda i: (0, i, 0)),
            pl.BlockSpec((NBLK, 1), lambda i: (i, 0)),
            pl.BlockSpec((1, D), lambda i: (0, 0)),
        ],
        out_specs=pl.BlockSpec((NBLK, K, D), lambda i: (i, 0, 0)),
        out_shape=jax.ShapeDtypeStruct((N, K, D), jnp.float32),
    )(agg3, dis, b2d)


def kernel(inputs, edge_index, W, b):
    row2d = edge_index[0].reshape(E // EB, EB)
    col2d = edge_index[1].reshape(E // EB, EB)
    y = _matmul(inputs.reshape(N, K * D), W)
    deg_part = _deg_partials(col2d)
    z3, dis = _z_and_dis(y, deg_part)
    agg = _aggregate(z3, row2d, col2d)
    return _finalize(agg, dis, b.reshape(1, D))


# speedup vs baseline: 1.2075x; 1.0390x over previous
import functools

import jax
import jax.numpy as jnp
from jax import lax
from jax.experimental import pallas as pl
from jax.experimental.pallas import tpu as pltpu
from jax.experimental.pallas import tpu_sc as plsc

N = 10000
E = 320000
D = 128
K = 4

NUM_CORES = 2
NUM_TILES = 16
EB = 80
ROWS_PER_TILE = N // NUM_TILES
EDGES_PER_TILE_A = E // (NUM_CORES * NUM_TILES)
NB_A = EDGES_PER_TILE_A // EB
EDGES_PER_TILE_C = E // NUM_TILES
NB_C = EDGES_PER_TILE_C // EB

NBLK = 2000
NGRID = N // NBLK


def _mesh():
    return plsc.VectorSubcoreMesh(core_axis_name="c", subcore_axis_name="s")


def _sc_params():
    return pltpu.CompilerParams(use_tc_tiling_on_sc=False)


def _deg_body(col_hbm, deg_hbm, acc, idx_v, ones_v, zrow_v):
    c = lax.axis_index("c")
    s = lax.axis_index("s")
    one16 = jnp.full((16,), 1.0, jnp.float32)
    zero16 = jnp.zeros((16,), jnp.float32)
    for r in range(EB):
        ones_v[r, :] = one16
    for r in range(NB_A):
        zrow_v[r, :] = zero16
    for j in range(ROWS_PER_TILE // NB_A):
        pltpu.sync_copy(zrow_v, acc.at[pl.ds(s * ROWS_PER_TILE + j * NB_A, NB_A)])
    plsc.subcore_barrier()
    base_row = c * (NUM_TILES * NB_A) + s * NB_A
    pltpu.sync_copy(col_hbm.at[pl.ds(base_row, NB_A)], idx_v)

    def body(i, carry):
        pltpu.sync_copy(ones_v, acc.at[idx_v.at[i]], add=True)
        return carry

    lax.fori_loop(0, NB_A, body, 0)
    plsc.subcore_barrier()
    pltpu.sync_copy(acc.at[pl.ds(s * ROWS_PER_TILE, ROWS_PER_TILE)],
                    deg_hbm.at[c, pl.ds(s * ROWS_PER_TILE, ROWS_PER_TILE)])


def _deg_partials(col2d):
    fn = pl.kernel(
        _deg_body,
        out_type=jax.ShapeDtypeStruct((NUM_CORES, N, 16), jnp.float32),
        mesh=_mesh(),
        compiler_params=_sc_params(),
        scratch_types=[
            pltpu.VMEM_SHARED((N, 16), jnp.float32),
            pltpu.VMEM((NB_A, EB), jnp.int32),
            pltpu.VMEM((EB, 16), jnp.float32),
            pltpu.VMEM((NB_A, 16), jnp.float32),
        ],
    )
    return fn(col2d)


def _mm_body(inp_ref, w_ref, y_ref):
    w = w_ref[...]
    for k in range(K):
        x = inp_ref[:, k * D:(k + 1) * D]
        y_ref[k, :, :] = jnp.dot(x, w, preferred_element_type=jnp.float32)


def _matmul(inputs2d, W):
    return pl.pallas_call(
        _mm_body,
        grid=(NGRID,),
        in_specs=[
            pl.BlockSpec((NBLK, K * D), lambda i: (i, 0)),
            pl.BlockSpec((D, D), lambda i: (0, 0)),
        ],
        out_specs=pl.BlockSpec((K, NBLK, D), lambda i: (0, i, 0)),
        out_shape=jax.ShapeDtypeStruct((K, N, D), jnp.float32),
    )(inputs2d, W)


def _zdis_body(y_ref, degp_ref, z_ref, dis_ref):
    d = degp_ref[...]
    deg = jnp.sum(d[0] + d[1], axis=1, keepdims=True) * (1.0 / 16.0) + 1.0
    dis = lax.rsqrt(deg)
    for k in range(K):
        z_ref[k, :, :] = y_ref[k, :, :] * dis
    dis_ref[...] = dis


def _z_and_dis(y, deg_part):
    return pl.pallas_call(
        _zdis_body,
        grid=(NGRID,),
        in_specs=[
            pl.BlockSpec((K, NBLK, D), lambda i: (0, i, 0)),
            pl.BlockSpec((NUM_CORES, NBLK, 16), lambda i: (0, i, 0)),
        ],
        out_specs=[
            pl.BlockSpec((K, NBLK, D), lambda i: (0, i, 0)),
            pl.BlockSpec((NBLK, 1), lambda i: (i, 0)),
        ],
        out_shape=[
            jax.ShapeDtypeStruct((K, N, D), jnp.float32),
            jax.ShapeDtypeStruct((N, 1), jnp.float32),
        ],
    )(y, deg_part)


CH = 125
NBUF = 3


def _agg_body(z_hbm, row_hbm, col_hbm, agg_hbm, acc, rowi_v, coli_v,
              r0, r1, r2, sg0, sg1, sg2, ss0, ss1, ss2):
    c = lax.axis_index("c")
    s = lax.axis_index("s")
    bufs = (r0, r1, r2)
    gsem = (sg0, sg1, sg2)
    ssem = (ss0, ss1, ss2)

    def run_chunk(kk, batch0, nb):
        pltpu.sync_copy(row_hbm.at[pl.ds(batch0, nb)], rowi_v.at[pl.ds(0, nb)])
        pltpu.sync_copy(col_hbm.at[pl.ds(batch0, nb)], coli_v.at[pl.ds(0, nb)])
        zk = z_hbm.at[kk]

        def g_start(g, b):
            pltpu.async_copy(zk.at[rowi_v.at[g]], bufs[b], gsem[b])

        def g_wait(g, b):
            pltpu.make_async_copy(zk.at[rowi_v.at[g]], bufs[b], gsem[b]).wait()

        def s_start(g, b):
            pltpu.async_copy(bufs[b], acc.at[coli_v.at[g]], ssem[b], add=True)

        def s_wait(g, b):
            pltpu.make_async_copy(bufs[b], acc.at[coli_v.at[g]], ssem[b]).wait()

        def emit(g, b, wait_prev, start_next):
            g_wait(g, b)
            s_start(g, b)
            if wait_prev:
                s_wait(g - 1, (b + 2) % NBUF)
            if start_next:
                g_start(g + 2, (b + 2) % NBUF)

        g_start(0, 0)
        g_start(1, 1)
        emit(0, 0, False, nb > 2)
        m3 = ((nb - 3) // NBUF) * NBUF

        def pipe(p, carry):
            g = 1 + NBUF * p
            for j in range(NBUF):
                emit(g + j, (1 + j) % NBUF, True, True)
            return carry

        lax.fori_loop(0, m3 // NBUF, pipe, 0)
        for g in range(1 + m3, nb):
            emit(g, g % NBUF, True, g + 2 < nb)
        s_wait(nb - 1, (nb - 1) % NBUF)

    for sl in range(2):
        kk = 2 * c + sl
        pltpu.sync_copy(z_hbm.at[kk, pl.ds(s * ROWS_PER_TILE, ROWS_PER_TILE)],
                        acc.at[pl.ds(s * ROWS_PER_TILE, ROWS_PER_TILE)])
        plsc.subcore_barrier()
        run_chunk(kk, s * NB_C, CH)
        run_chunk(kk, s * NB_C + CH, NB_C - CH)
        plsc.subcore_barrier()
        pltpu.sync_copy(acc.at[pl.ds(s * ROWS_PER_TILE, ROWS_PER_TILE)],
                        agg_hbm.at[kk, pl.ds(s * ROWS_PER_TILE, ROWS_PER_TILE)])


def _aggregate(z, row2d, col2d):
    fn = pl.kernel(
        _agg_body,
        out_type=jax.ShapeDtypeStruct((K, N, D), jnp.float32),
        mesh=_mesh(),
        compiler_params=_sc_params(),
        scratch_types=[
            pltpu.VMEM_SHARED((N, D), jnp.float32),
            pltpu.VMEM((CH, EB), jnp.int32),
            pltpu.VMEM((CH, EB), jnp.int32),
            pltpu.VMEM((EB, D), jnp.float32),
            pltpu.VMEM((EB, D), jnp.float32),
            pltpu.VMEM((EB, D), jnp.float32),
            pltpu.SemaphoreType.DMA,
            pltpu.SemaphoreType.DMA,
            pltpu.SemaphoreType.DMA,
            pltpu.SemaphoreType.DMA,
            pltpu.SemaphoreType.DMA,
            pltpu.SemaphoreType.DMA,
        ],
    )
    return fn(z, row2d, col2d)


def _out_body(agg_ref, dis_ref, b_ref, o_ref):
    dis = dis_ref[...]
    bb = b_ref[...]
    for k in range(K):
        o_ref[:, k, :] = jnp.maximum(agg_ref[k, :, :] * dis + bb, 0.0)


def _finalize(agg3, dis, b2d):
    return pl.pallas_call(
        _out_body,
        grid=(NGRID,),
        in_specs=[
            pl.BlockSpec((K, NBLK, D), lambda i: (0, i, 0)),
            pl.BlockSpec((NBLK, 1), lambda i: (i, 0)),
            pl.BlockSpec((1, D), lambda i: (0, 0)),
        ],
        out_specs=pl.BlockSpec((NBLK, K, D), lambda i: (i, 0, 0)),
        out_shape=jax.ShapeDtypeStruct((N, K, D), jnp.float32),
    )(agg3, dis, b2d)


def kernel(inputs, edge_index, W, b):
    row2d = edge_index[0].reshape(E // EB, EB)
    col2d = edge_index[1].reshape(E // EB, EB)
    y = _matmul(inputs.reshape(N, K * D), W)
    deg_part = _deg_partials(col2d)
    z3, dis = _z_and_dis(y, deg_part)
    agg = _aggregate(z3, row2d, col2d)
    return _finalize(agg, dis, b.reshape(1, D))


# speedup vs baseline: 1.2117x; 1.0034x over previous
import functools

import jax
import jax.numpy as jnp
from jax import lax
from jax.experimental import pallas as pl
from jax.experimental.pallas import tpu as pltpu
from jax.experimental.pallas import tpu_sc as plsc

N = 10000
E = 320000
D = 128
K = 4

NUM_CORES = 2
NUM_TILES = 16
EB = 80
ROWS_PER_TILE = N // NUM_TILES
EDGES_PER_TILE_A = E // (NUM_CORES * NUM_TILES)
NB_A = EDGES_PER_TILE_A // EB
EDGES_PER_TILE_C = E // NUM_TILES
NB_C = EDGES_PER_TILE_C // EB

NBLK = 5000
NGRID = N // NBLK


def _mesh():
    return plsc.VectorSubcoreMesh(core_axis_name="c", subcore_axis_name="s")


def _sc_params():
    return pltpu.CompilerParams(use_tc_tiling_on_sc=False)


def _deg_body(col_hbm, deg_hbm, acc, idx_v, ones_v, zrow_v):
    c = lax.axis_index("c")
    s = lax.axis_index("s")
    one16 = jnp.full((16,), 1.0, jnp.float32)
    zero16 = jnp.zeros((16,), jnp.float32)
    for r in range(EB):
        ones_v[r, :] = one16
    for r in range(NB_A):
        zrow_v[r, :] = zero16
    for j in range(ROWS_PER_TILE // NB_A):
        pltpu.sync_copy(zrow_v, acc.at[pl.ds(s * ROWS_PER_TILE + j * NB_A, NB_A)])
    plsc.subcore_barrier()
    base_row = c * (NUM_TILES * NB_A) + s * NB_A
    pltpu.sync_copy(col_hbm.at[pl.ds(base_row, NB_A)], idx_v)

    def body(i, carry):
        pltpu.sync_copy(ones_v, acc.at[idx_v.at[i]], add=True)
        return carry

    lax.fori_loop(0, NB_A, body, 0)
    plsc.subcore_barrier()
    pltpu.sync_copy(acc.at[pl.ds(s * ROWS_PER_TILE, ROWS_PER_TILE)],
                    deg_hbm.at[c, pl.ds(s * ROWS_PER_TILE, ROWS_PER_TILE)])


def _deg_partials(col2d):
    fn = pl.kernel(
        _deg_body,
        out_type=jax.ShapeDtypeStruct((NUM_CORES, N, 16), jnp.float32),
        mesh=_mesh(),
        compiler_params=_sc_params(),
        scratch_types=[
            pltpu.VMEM_SHARED((N, 16), jnp.float32),
            pltpu.VMEM((NB_A, EB), jnp.int32),
            pltpu.VMEM((EB, 16), jnp.float32),
            pltpu.VMEM((NB_A, 16), jnp.float32),
        ],
    )
    return fn(col2d)


def _mm_body(inp_ref, w_ref, y_ref):
    w = w_ref[...]
    for k in range(K):
        x = inp_ref[:, k * D:(k + 1) * D]
        y_ref[k, :, :] = jnp.dot(x, w, preferred_element_type=jnp.float32)


def _matmul(inputs2d, W):
    return pl.pallas_call(
        _mm_body,
        grid=(NGRID,),
        in_specs=[
            pl.BlockSpec((NBLK, K * D), lambda i: (i, 0)),
            pl.BlockSpec((D, D), lambda i: (0, 0)),
        ],
        out_specs=pl.BlockSpec((K, NBLK, D), lambda i: (0, i, 0)),
        out_shape=jax.ShapeDtypeStruct((K, N, D), jnp.float32),
    )(inputs2d, W)


def _zdis_body(y_ref, degp_ref, z_ref, dis_ref):
    d = degp_ref[...]
    deg = jnp.sum(d[0] + d[1], axis=1, keepdims=True) * (1.0 / 16.0) + 1.0
    dis = lax.rsqrt(deg)
    for k in range(K):
        z_ref[k, :, :] = y_ref[k, :, :] * dis
    dis_ref[...] = dis


def _z_and_dis(y, deg_part):
    return pl.pallas_call(
        _zdis_body,
        grid=(NGRID,),
        in_specs=[
            pl.BlockSpec((K, NBLK, D), lambda i: (0, i, 0)),
            pl.BlockSpec((NUM_CORES, NBLK, 16), lambda i: (0, i, 0)),
        ],
        out_specs=[
            pl.BlockSpec((K, NBLK, D), lambda i: (0, i, 0)),
            pl.BlockSpec((NBLK, 1), lambda i: (i, 0)),
        ],
        out_shape=[
            jax.ShapeDtypeStruct((K, N, D), jnp.float32),
            jax.ShapeDtypeStruct((N, 1), jnp.float32),
        ],
    )(y, deg_part)


CH = 125
NBUF = 3


def _agg_body(z_hbm, row_hbm, col_hbm, agg_hbm, acc, rowi_v, coli_v,
              r0, r1, r2, sg0, sg1, sg2, ss0, ss1, ss2):
    c = lax.axis_index("c")
    s = lax.axis_index("s")
    bufs = (r0, r1, r2)
    gsem = (sg0, sg1, sg2)
    ssem = (ss0, ss1, ss2)

    def run_chunk(kk, batch0, nb):
        pltpu.sync_copy(row_hbm.at[pl.ds(batch0, nb)], rowi_v.at[pl.ds(0, nb)])
        pltpu.sync_copy(col_hbm.at[pl.ds(batch0, nb)], coli_v.at[pl.ds(0, nb)])
        zk = z_hbm.at[kk]

        def g_start(g, b):
            pltpu.async_copy(zk.at[rowi_v.at[g]], bufs[b], gsem[b])

        def g_wait(g, b):
            pltpu.make_async_copy(zk.at[rowi_v.at[g]], bufs[b], gsem[b]).wait()

        def s_start(g, b):
            pltpu.async_copy(bufs[b], acc.at[coli_v.at[g]], ssem[b], add=True)

        def s_wait(g, b):
            pltpu.make_async_copy(bufs[b], acc.at[coli_v.at[g]], ssem[b]).wait()

        def emit(g, b, wait_prev, start_next):
            g_wait(g, b)
            s_start(g, b)
            if wait_prev:
                s_wait(g - 1, (b + 2) % NBUF)
            if start_next:
                g_start(g + 2, (b + 2) % NBUF)

        g_start(0, 0)
        g_start(1, 1)
        emit(0, 0, False, nb > 2)
        m3 = ((nb - 3) // NBUF) * NBUF

        def pipe(p, carry):
            g = 1 + NBUF * p
            for j in range(NBUF):
                emit(g + j, (1 + j) % NBUF, True, True)
            return carry

        lax.fori_loop(0, m3 // NBUF, pipe, 0)
        for g in range(1 + m3, nb):
            emit(g, g % NBUF, True, g + 2 < nb)
        s_wait(nb - 1, (nb - 1) % NBUF)

    for sl in range(2):
        kk = 2 * c + sl
        pltpu.sync_copy(z_hbm.at[kk, pl.ds(s * ROWS_PER_TILE, ROWS_PER_TILE)],
                        acc.at[pl.ds(s * ROWS_PER_TILE, ROWS_PER_TILE)])
        plsc.subcore_barrier()
        run_chunk(kk, s * NB_C, CH)
        run_chunk(kk, s * NB_C + CH, NB_C - CH)
        plsc.subcore_barrier()
        pltpu.sync_copy(acc.at[pl.ds(s * ROWS_PER_TILE, ROWS_PER_TILE)],
                        agg_hbm.at[kk, pl.ds(s * ROWS_PER_TILE, ROWS_PER_TILE)])


def _aggregate(z, row2d, col2d):
    fn = pl.kernel(
        _agg_body,
        out_type=jax.ShapeDtypeStruct((K, N, D), jnp.float32),
        mesh=_mesh(),
        compiler_params=_sc_params(),
        scratch_types=[
            pltpu.VMEM_SHARED((N, D), jnp.float32),
            pltpu.VMEM((CH, EB), jnp.int32),
            pltpu.VMEM((CH, EB), jnp.int32),
            pltpu.VMEM((EB, D), jnp.float32),
            pltpu.VMEM((EB, D), jnp.float32),
            pltpu.VMEM((EB, D), jnp.float32),
            pltpu.SemaphoreType.DMA,
            pltpu.SemaphoreType.DMA,
            pltpu.SemaphoreType.DMA,
            pltpu.SemaphoreType.DMA,
            pltpu.SemaphoreType.DMA,
            pltpu.SemaphoreType.DMA,
        ],
    )
    return fn(z, row2d, col2d)


def _out_body(agg_ref, dis_ref, b_ref, o_ref):
    dis = dis_ref[...]
    bb = b_ref[...]
    for k in range(K):
        o_ref[:, k, :] = jnp.maximum(agg_ref[k, :, :] * dis + bb, 0.0)


def _finalize(agg3, dis, b2d):
    return pl.pallas_call(
        _out_body,
        grid=(NGRID,),
        in_specs=[
            pl.BlockSpec((K, NBLK, D), lambda i: (0, i, 0)),
            pl.BlockSpec((NBLK, 1), lambda i: (i, 0)),
            pl.BlockSpec((1, D), lambda i: (0, 0)),
        ],
        out_specs=pl.BlockSpec((NBLK, K, D), lambda i: (i, 0, 0)),
        out_shape=jax.ShapeDtypeStruct((N, K, D), jnp.float32),
    )(agg3, dis, b2d)


def kernel(inputs, edge_index, W, b):
    row2d = edge_index[0].reshape(E // EB, EB)
    col2d = edge_index[1].reshape(E // EB, EB)
    y = _matmul(inputs.reshape(N, K * D), W)
    deg_part = _deg_partials(col2d)
    z3, dis = _z_and_dis(y, deg_part)
    agg = _aggregate(z3, row2d, col2d)
    return _finalize(agg, dis, b.reshape(1, D))
